# baseline (speedup 1.0000x reference)
import functools

import jax
import jax.numpy as jnp
from jax import lax
from jax.experimental import pallas as pl
from jax.experimental.pallas import tpu as pltpu
from jax.experimental.pallas import tpu_sc as plsc

N = 10000
D = 128
G = 128
E = 320000

NP = 10240
E2 = 327680
SENT = 10000
NC = 2
NS = 16
CH = 128
CL = 64

f32 = jnp.float32
i32 = jnp.int32

def _sc_mesh():
    return plsc.VectorSubcoreMesh(
        core_axis_name="c", subcore_axis_name="s",
        num_cores=NC, num_subcores=NS)


def _k1_body(x_ref, w_ref, b_ref, bcol_ref, brow_ref, bprev_ref,
             h_ref, roots_ref, sizes_ref, isroot_ref):
    i = pl.program_id(0)
    h = jnp.dot(x_ref[...], w_ref[...], preferred_element_type=f32) + b_ref[...]
    h_ref[...] = jnp.where(h >= 0, h, 0.01 * h)

    bcol = jnp.broadcast_to(bcol_ref[...], (1000, G))
    lane = lax.broadcasted_iota(i32, (1000, G), 1)
    roots_p = jnp.sum((bcol < lane).astype(f32), axis=0, keepdims=True)
    sizes_p = jnp.sum((bcol == lane).astype(f32), axis=0, keepdims=True)

    @pl.when(i == 0)
    def _():
        roots_ref[...] = roots_p
        sizes_ref[...] = sizes_p

    @pl.when(i > 0)
    def _():
        roots_ref[...] += roots_p
        sizes_ref[...] += sizes_p

    isroot_ref[0] = (brow_ref[0] != bprev_ref[0]).astype(i32)


def _run_k1(x, adapter_w, adapter_b, batch):
    bcol = batch.reshape(N, 1)
    brow = batch.reshape(10, 1, 1000)
    bprev = jnp.concatenate([batch[:1] - 1, batch[:-1]]).reshape(10, 1, 1000)
    return pl.pallas_call(
        _k1_body,
        grid=(10,),
        in_specs=[
            pl.BlockSpec((1000, D), lambda i: (i, 0)),
            pl.BlockSpec((D, D), lambda i: (0, 0)),
            pl.BlockSpec((1, D), lambda i: (0, 0)),
            pl.BlockSpec((1000, 1), lambda i: (i, 0)),
            pl.BlockSpec((1, 1, 1000), lambda i: (i, 0, 0)),
            pl.BlockSpec((1, 1, 1000), lambda i: (i, 0, 0)),
        ],
        out_specs=[
            pl.BlockSpec((1000, D), lambda i: (i, 0)),
            pl.BlockSpec((1, G), lambda i: (0, 0)),
            pl.BlockSpec((1, G), lambda i: (0, 0)),
            pl.BlockSpec((1, 1, 1000), lambda i: (i, 0, 0)),
        ],
        out_shape=[
            jax.ShapeDtypeStruct((N, D), f32),
            jax.ShapeDtypeStruct((1, G), f32),
            jax.ShapeDtypeStruct((1, G), f32),
            jax.ShapeDtypeStruct((10, 1, 1000), i32),
        ],
    )(x, adapter_w, adapter_b.reshape(1, D), bcol, brow, bprev)


def _k2_body(src_hbm, dst_hbm, batch_hbm, isroot_hbm,
             pdegt_hbm, pdegb_hbm, pcnt_hbm,
             sidx, didx, batch_vm, isroot_vm, degt, degb, cnt):
    c = lax.axis_index("c")
    s = lax.axis_index("s")
    w = c * NS + s

    pltpu.sync_copy(src_hbm.at[w], sidx)
    pltpu.sync_copy(dst_hbm.at[w], didx)
    pltpu.sync_copy(batch_hbm, batch_vm)
    pltpu.sync_copy(isroot_hbm, isroot_vm)

    z16 = jnp.zeros((16,), f32)

    def zero_deg(j, _):
        degt[pl.ds(j * 16, 16)] = z16
        degb[pl.ds(j * 16, 16)] = z16
        return None
    lax.fori_loop(0, NP // 16, zero_deg, None)

    def zero_cnt(j, _):
        cnt[pl.ds(j * 16, 16)] = z16
        return None
    lax.fori_loop(0, 16, zero_cnt, None)

    ones = jnp.ones((16,), f32)
    one_i = jnp.ones((16,), i32)

    def chunk(t, _):
        for k in range(8):
            sv = sidx[t, pl.ds(k * 16, 16)]
            dv = didx[t, pl.ds(k * 16, 16)]
            bs = plsc.load_gather(batch_vm, [sv])
            bd = plsc.load_gather(batch_vm, [dv])
            rt = plsc.load_gather(isroot_vm, [sv])
            within = jnp.where((bs == bd) & (rt == one_i), 1.0, 0.0)
            plsc.addupdate_scatter(degt, [dv], ones)
            plsc.addupdate_scatter(degb, [sv], ones)
            plsc.addupdate_scatter(cnt, [bs], within)
        return None
    lax.fori_loop(0, 80, chunk, None)

    pltpu.sync_copy(degt, pdegt_hbm.at[w])
    pltpu.sync_copy(degb, pdegb_hbm.at[w])
    pltpu.sync_copy(cnt, pcnt_hbm.at[w])


def _run_k2(src_st, dst_st, batch_pad, isroot_pad):
    k = pl.kernel(
        _k2_body,
        out_type=[
            jax.ShapeDtypeStruct((32, NP), f32),
            jax.ShapeDtypeStruct((32, NP), f32),
            jax.ShapeDtypeStruct((32, 256), f32),
        ],
        mesh=_sc_mesh(),
        compiler_params=pltpu.CompilerParams(needs_layout_passes=False),
        scratch_types=[
            pltpu.VMEM((80, CH), i32),
            pltpu.VMEM((80, CH), i32),
            pltpu.VMEM((NP,), i32),
            pltpu.VMEM((NP,), i32),
            pltpu.VMEM((NP,), f32),
            pltpu.VMEM((NP,), f32),
            pltpu.VMEM((256,), f32),
        ],
    )
    return k(src_st, dst_st, batch_pad, isroot_pad)


def _ln_prompt(rf, w1, b1, g, be, w2, b2):
    t = jnp.dot(rf, w1, preferred_element_type=f32) + b1
    mu = jnp.mean(t, axis=-1, keepdims=True)
    var = jnp.mean((t - mu) ** 2, axis=-1, keepdims=True)
    t = (t - mu) * lax.rsqrt(var + 1e-5) * g + be
    return jnp.dot(jnp.tanh(t), w2, preferred_element_type=f32) + b2


def _k3a_body(pdegt_ref, pdegb_ref, pcnt_ref, sizes_ref, roots_ref,
              h_ref, brow_ref,
              p1w1, p1b1, p1g, p1be, p1w2, p1b2,
              p2w1, p2b1, p2g, p2be, p2w2, p2b2,
              dist_ref, disb_ref, pm_ref, pa_ref, alpha_ref, rf_acc):
    i = pl.program_id(0)
    degt = jnp.sum(pdegt_ref[...], axis=0, keepdims=True) + 1.0
    dist_ref[...] = lax.rsqrt(degt)
    degb = jnp.sum(pdegb_ref[...], axis=0, keepdims=True) + 1.0
    disb_ref[...] = lax.rsqrt(degb)

    gid = (lax.broadcasted_iota(i32, (G, 1000), 1) + i * 1000).astype(f32)
    r1h = jnp.broadcast_to(roots_ref[...], (G, 1000))
    rmat = (gid == r1h).astype(f32)
    rfp = jnp.dot(rmat, h_ref[...], preferred_element_type=f32)

    @pl.when(i == 0)
    def _():
        rf_acc[...] = rfp

    @pl.when(i > 0)
    def _():
        rf_acc[...] += rfp

    @pl.when(i == 9)
    def _():
        rf = rf_acc[...]
        pm_ref[...] = _ln_prompt(rf, p1w1[...], p1b1[...], p1g[...],
                                 p1be[...], p1w2[...], p1b2[...])
        pa_ref[...] = _ln_prompt(rf, p2w1[...], p2b1[...], p2g[...],
                                 p2be[...], p2w2[...], p2b2[...])
        cntv = jnp.sum(pcnt_ref[...], axis=0, keepdims=True)[:, :G]
        one_level = cntv / jnp.maximum(sizes_ref[...], 1.0)
        ag = jax.nn.sigmoid((one_level - 0.5) / 0.1)
        eye = (lax.broadcasted_iota(i32, (G, G), 0)
               == lax.broadcasted_iota(i32, (G, G), 1)).astype(f32)
        acol = lax.dot_general(eye, ag, (((0,), (1,)), ((), ())),
                               preferred_element_type=f32)
        alpha_ref[...] = jnp.broadcast_to(acol, (G, G))


def _run_k3a(pdegt, pdegb, pcnt, sizes, roots, h, batch, p):
    roots_col = roots.reshape(G, 1)
    brow = batch.reshape(10, 1, 1000)
    const = lambda i: (0, 0)
    pshape = [(D, 128), (1, 128), (1, 128), (1, 128), (128, D), (1, D)]
    pspecs = [pl.BlockSpec(s, const) for s in pshape] * 2
    pargs = [p['p1_W1'], p['p1_b1'].reshape(1, 128), p['p1_g'].reshape(1, 128),
             p['p1_be'].reshape(1, 128), p['p1_W2'], p['p1_b2'].reshape(1, D),
             p['p2_W1'], p['p2_b1'].reshape(1, 128), p['p2_g'].reshape(1, 128),
             p['p2_be'].reshape(1, 128), p['p2_W2'], p['p2_b2'].reshape(1, D)]
    return pl.pallas_call(
        _k3a_body,
        grid=(10,),
        in_specs=[
            pl.BlockSpec((32, 1024), lambda i: (0, i)),
            pl.BlockSpec((32, 1024), lambda i: (0, i)),
            pl.BlockSpec((32, 256), const),
            pl.BlockSpec((1, G), const),
            pl.BlockSpec((G, 1), const),
            pl.BlockSpec((1000, D), lambda i: (i, 0)),
            pl.BlockSpec((1, 1, 1000), lambda i: (i, 0, 0)),
        ] + pspecs,
        out_specs=[
            pl.BlockSpec((1, 1024), lambda i: (0, i)),
            pl.BlockSpec((1, 1024), lambda i: (0, i)),
            pl.BlockSpec((G, D), const),
            pl.BlockSpec((G, D), const),
            pl.BlockSpec((G, G), const),
        ],
        out_shape=[
            jax.ShapeDtypeStruct((1, NP), f32),
            jax.ShapeDtypeStruct((1, NP), f32),
            jax.ShapeDtypeStruct((G, D), f32),
            jax.ShapeDtypeStruct((G, D), f32),
            jax.ShapeDtypeStruct((G, G), f32),
        ],
        scratch_shapes=[pltpu.VMEM((G, D), f32)],
    )(pdegt, pdegb, pcnt, sizes, roots_col, h, brow, *pargs)


def _k3b_body(h_ref, brow_ref, dtc_ref, dbc_ref, pm_ref, pa_ref, alpha_ref,
              tdw_ref, buw_ref, mpt_ref, mpb_ref):
    bt = (lax.broadcasted_iota(i32, (G, 1000), 0)
          == jnp.broadcast_to(brow_ref[0], (G, 1000))).astype(f32)
    cdims = (((0,), (0,)), ((), ()))
    pm_n = lax.dot_general(bt, pm_ref[...], cdims, preferred_element_type=f32)
    pa_n = lax.dot_general(bt, pa_ref[...], cdims, preferred_element_type=f32)
    al_n = lax.dot_general(bt, alpha_ref[...], cdims, preferred_element_type=f32)
    h = h_ref[...]
    t1 = h * pm_n
    z = t1 + al_n * (h + pa_n - t1)
    mpt_ref[...] = jnp.dot(z, tdw_ref[...], preferred_element_type=f32) * dtc_ref[...]
    mpb_ref[...] = jnp.dot(z, buw_ref[...], preferred_element_type=f32) * dbc_ref[...]


def _run_k3b(h, batch, dist_col, disb_col, pm_g, pa_g, alpha_bcg, td_w1, bu_w1):
    const = lambda i: (0, 0)
    return pl.pallas_call(
        _k3b_body,
        grid=(10,),
        in_specs=[
            pl.BlockSpec((1000, D), lambda i: (i, 0)),
            pl.BlockSpec((1, 1, 1000), lambda i: (i, 0, 0)),
            pl.BlockSpec((1000, 1), lambda i: (i, 0)),
            pl.BlockSpec((1000, 1), lambda i: (i, 0)),
            pl.BlockSpec((G, D), const),
            pl.BlockSpec((G, D), const),
            pl.BlockSpec((G, G), const),
            pl.BlockSpec((D, D), const),
            pl.BlockSpec((D, D), const),
        ],
        out_specs=[
            pl.BlockSpec((1000, D), lambda i: (i, 0)),
            pl.BlockSpec((1000, D), lambda i: (i, 0)),
        ],
        out_shape=[
            jax.ShapeDtypeStruct((N, D), f32),
            jax.ShapeDtypeStruct((N, D), f32),
        ],
    )(h, batch.reshape(10, 1, 1000), dist_col, disb_col, pm_g, pa_g,
      alpha_bcg, td_w1, bu_w1)


def _layer_body(mp_hbm, gidx_hbm, didx_hbm, agg_hbm,
                sidxb, didxb, buf0, buf1, buf2, buf3, acc,
                is0, is1, gs0, gs1, gs2, gs3, ss0, ss1, ss2, ss3):
    c = lax.axis_index("c")
    s = lax.axis_index("s")

    z16 = jnp.zeros((16,), f32)

    def zero_buf(r, _):
        for k in range(8):
            buf0[r, pl.ds(k * 16, 16)] = z16
        return None
    lax.fori_loop(0, CL, zero_buf, None)

    def zero_acc(j, _):
        pltpu.sync_copy(buf0, acc.at[pl.ds(s * 640 + j * CL, CL)])
        return None
    lax.fori_loop(0, 10, zero_acc, None)
    plsc.subcore_barrier()

    bufs = (buf0, buf1, buf2, buf3)
    gsems = (gs0, gs1, gs2, gs3)
    ssems = (ss0, ss1, ss2, ss3)
    isems = (is0, is1)

    def idx_issue(blk, p):
        pltpu.async_copy(gidx_hbm.at[c, s, blk], sidxb.at[p], isems[p])
        pltpu.async_copy(didx_hbm.at[c, s, blk], didxb.at[p], isems[p])

    def idx_wait(blk, p):
        pltpu.make_async_copy(gidx_hbm.at[c, s, blk], sidxb.at[p], isems[p]).wait()
        pltpu.make_async_copy(didx_hbm.at[c, s, blk], didxb.at[p], isems[p]).wait()

    def g_issue(p2, j2, q):
        pltpu.async_copy(mp_hbm.at[sidxb.at[p2, j2]], bufs[q], gsems[q])

    def g_wait(p2, j2, q):
        pltpu.make_async_copy(mp_hbm.at[sidxb.at[p2, j2]], bufs[q], gsems[q]).wait()

    def s_issue(p2, j2, q):
        pltpu.async_copy(bufs[q], acc.at[didxb.at[p2, j2]], ssems[q], add=True)

    def s_wait(q):
        pltpu.make_async_copy(bufs[q], acc.at[didxb.at[0, 0]], ssems[q]).wait()

    idx_issue(0, 0)
    idx_issue(1, 1)
    idx_wait(0, 0)
    g_issue(0, 0, 0)
    g_issue(0, 1, 1)

    def dblock(db, _):
        for p in range(2):
            b = 2 * db + p
            for j in range(16):
                q = j % 4
                if j == 2:
                    @pl.when((b > 0) & (b < 19))
                    def _():
                        idx_issue(b + 1, 1 - p)
                if j == 14:
                    @pl.when(b + 1 < 20)
                    def _():
                        idx_wait(b + 1, 1 - p)

                g_wait(p, j, q)
                s_issue(p, j, q)
                qd = (j + 2) % 4
                if j < 2:
                    @pl.when(b > 0)
                    def _():
                        s_wait(qd)
                else:
                    s_wait(qd)
                if j < 14:
                    g_issue(p, j + 2, qd)
                else:
                    @pl.when(b < 19)
                    def _():
                        g_issue(1 - p, j - 14, qd)
        return None
    lax.fori_loop(0, 10, dblock, None)

    s_wait(2)
    s_wait(3)

    plsc.subcore_barrier()

    def writeback(j, _):
        pltpu.sync_copy(acc.at[pl.ds(s * 640 + j * CL, CL)], buf0)
        pltpu.sync_copy(buf0, agg_hbm.at[c, pl.ds(s * 640 + j * CL, CL)])
        return None
    lax.fori_loop(0, 10, writeback, None)


def _run_layer(mp_all, gidx, didx):
    k = pl.kernel(
        _layer_body,
        out_type=jax.ShapeDtypeStruct((NC, NP, D), f32),
        mesh=_sc_mesh(),
        compiler_params=pltpu.CompilerParams(needs_layout_passes=False),
        scratch_types=[
            pltpu.VMEM((2, 16, CL), i32),
            pltpu.VMEM((2, 16, CL), i32),
            pltpu.VMEM((CL, D), f32),
            pltpu.VMEM((CL, D), f32),
            pltpu.VMEM((CL, D), f32),
            pltpu.VMEM((CL, D), f32),
            pltpu.VMEM_SHARED((NP, D), f32),
            pltpu.SemaphoreType.DMA,
            pltpu.SemaphoreType.DMA,
            pltpu.SemaphoreType.DMA,
            pltpu.SemaphoreType.DMA,
            pltpu.SemaphoreType.DMA,
            pltpu.SemaphoreType.DMA,
            pltpu.SemaphoreType.DMA,
            pltpu.SemaphoreType.DMA,
            pltpu.SemaphoreType.DMA,
            pltpu.SemaphoreType.DMA,
        ],
    )
    return k(mp_all, gidx, didx)


def _k5_body(aggt_ref, mpt_ref, aggb_ref, mpb_ref, dtc_ref, dbc_ref,
             tb1_ref, bb1_ref, tdw2_ref, buw2_ref, mpt2_ref, mpb2_ref):
    dt = dtc_ref[...]
    db = dbc_ref[...]
    h1t = dt * (aggt_ref[...] + mpt_ref[...]) + tb1_ref[...]
    h1t = jnp.maximum(h1t, 0.0)
    h1b = db * (aggb_ref[...] + mpb_ref[...]) + bb1_ref[...]
    h1b = jnp.maximum(h1b, 0.0)
    mpt2_ref[...] = jnp.dot(h1t, tdw2_ref[...], preferred_element_type=f32) * dt
    mpb2_ref[...] = jnp.dot(h1b, buw2_ref[...], preferred_element_type=f32) * db


def _run_k5(aggt, mpt, aggb, mpb, dist_col, disb_col, tb1, bb1, tdw2, buw2):
    const = lambda i: (0, 0)
    row = lambda i: (i, 0)
    return pl.pallas_call(
        _k5_body,
        grid=(10,),
        in_specs=[
            pl.BlockSpec((1000, D), row),
            pl.BlockSpec((1000, D), row),
            pl.BlockSpec((1000, D), row),
            pl.BlockSpec((1000, D), row),
            pl.BlockSpec((1000, 1), row),
            pl.BlockSpec((1000, 1), row),
            pl.BlockSpec((1, D), const),
            pl.BlockSpec((1, D), const),
            pl.BlockSpec((D, D), const),
            pl.BlockSpec((D, D), const),
        ],
        out_specs=[
            pl.BlockSpec((1000, D), row),
            pl.BlockSpec((1000, D), row),
        ],
        out_shape=[
            jax.ShapeDtypeStruct((N, D), f32),
            jax.ShapeDtypeStruct((N, D), f32),
        ],
    )(aggt, mpt, aggb, mpb, dist_col, disb_col,
      tb1.reshape(1, D), bb1.reshape(1, D), tdw2, buw2)


def _k7_body(aggt_ref, mpt_ref, aggb_ref, mpb_ref, dtc_ref, dbc_ref,
             brow_ref, tb2_ref, bb2_ref, pw1_ref, pb1_ref, pw2_ref, pb2_ref,
             out_ref, poolt, poolb):
    i = pl.program_id(0)
    q1 = dtc_ref[...] * (aggt_ref[...] + mpt_ref[...]) + tb2_ref[...]
    q2 = dbc_ref[...] * (aggb_ref[...] + mpb_ref[...]) + bb2_ref[...]
    bt = (lax.broadcasted_iota(i32, (G, 1000), 0)
          == jnp.broadcast_to(brow_ref[0], (G, 1000))).astype(f32)
    pt = jnp.dot(bt, q1, preferred_element_type=f32)
    pb = jnp.dot(bt, q2, preferred_element_type=f32)

    @pl.when(i == 0)
    def _():
        poolt[...] = pt
        poolb[...] = pb

    @pl.when(i > 0)
    def _():
        poolt[...] += pt
        poolb[...] += pb

    @pl.when(i == 9)
    def _():
        hc = jnp.concatenate([poolb[...], poolt[...]], axis=1)
        r = jnp.maximum(
            jnp.dot(hc, pw1_ref[...], preferred_element_type=f32) + pb1_ref[...],
            0.0)
        out_ref[...] = (jnp.dot(r, pw2_ref[...], preferred_element_type=f32)
                        + pb2_ref[...])


def _run_k7(aggt, mpt, aggb, mpb, dist_col, disb_col, batch, p):
    const = lambda i: (0, 0)
    row = lambda i: (i, 0)
    return pl.pallas_call(
        _k7_body,
        grid=(10,),
        in_specs=[
            pl.BlockSpec((1000, D), row),
            pl.BlockSpec((1000, D), row),
            pl.BlockSpec((1000, D), row),
            pl.BlockSpec((1000, D), row),
            pl.BlockSpec((1000, 1), row),
            pl.BlockSpec((1000, 1), row),
            pl.BlockSpec((1, 1, 1000), lambda i: (i, 0, 0)),
            pl.BlockSpec((1, D), const),
            pl.BlockSpec((1, D), const),
            pl.BlockSpec((2 * D, 256), const),
            pl.BlockSpec((1, 256), const),
            pl.BlockSpec((256, D), const),
            pl.BlockSpec((1, D), const),
        ],
        out_specs=pl.BlockSpec((G, D), const),
        out_shape=jax.ShapeDtypeStruct((G, D), f32),
        scratch_shapes=[pltpu.VMEM((G, D), f32), pltpu.VMEM((G, D), f32)],
    )(aggt, mpt, aggb, mpb, dist_col, disb_col, batch.reshape(10, 1, 1000),
      p['td_b2'].reshape(1, D), p['bu_b2'].reshape(1, D),
      p['proj_W1'], p['proj_b1'].reshape(1, 256),
      p['proj_W2'], p['proj_b2'].reshape(1, D))


def kernel(x, edge_index, batch, params):
    p = params
    src = edge_index[0]
    dst = edge_index[1]
    pad = jnp.full((E2 - E,), SENT, i32)
    srcp = jnp.concatenate([src, pad])
    dstp = jnp.concatenate([dst, pad])

    src_st = srcp.reshape(32, 80, CH)
    dst_st = dstp.reshape(32, 80, CH)
    gidx = jnp.stack([srcp, dstp + NP]).reshape(NC, NS, 20, 16, CL)
    didx = jnp.stack([dstp, srcp]).reshape(NC, NS, 20, 16, CL)

    batch_pad = jnp.concatenate([batch, jnp.full((NP - N,), G, i32)])

    h, roots, sizes, isroot = _run_k1(x, p['adapter_W'], p['adapter_b'], batch)
    isroot_pad = jnp.concatenate(
        [isroot.reshape(N), jnp.zeros((NP - N,), i32)])

    pdegt, pdegb, pcnt = _run_k2(src_st, dst_st, batch_pad, isroot_pad)

    dist_full, disb_full, pm_g, pa_g, alpha_bcg = _run_k3a(
        pdegt, pdegb, pcnt, sizes, roots, h, batch, p)
    dist_col = dist_full[0, :N].reshape(N, 1)
    disb_col = disb_full[0, :N].reshape(N, 1)

    mp1t, mp1b = _run_k3b(h, batch, dist_col, disb_col, pm_g, pa_g,
                          alpha_bcg, p['td_W1'], p['bu_W1'])

    zpad = jnp.zeros((NP - N, D), f32)
    mp_all1 = jnp.concatenate([mp1t, zpad, mp1b, zpad]).reshape(NC * NP, D)
    agg1 = _run_layer(mp_all1, gidx, didx)

    mp2t, mp2b = _run_k5(agg1[0, :N], mp1t, agg1[1, :N], mp1b,
                         dist_col, disb_col, p['td_b1'], p['bu_b1'],
                         p['td_W2'], p['bu_W2'])

    mp_all2 = jnp.concatenate([mp2t, zpad, mp2b, zpad]).reshape(NC * NP, D)
    agg2 = _run_layer(mp_all2, gidx, didx)

    return _run_k7(agg2[0, :N], mp2t, agg2[1, :N], mp2b,
                   dist_col, disb_col, batch, p)


# speedup vs baseline: 1.1886x; 1.1886x over previous
import functools

import jax
import jax.numpy as jnp
from jax import lax
from jax.experimental import pallas as pl
from jax.experimental.pallas import tpu as pltpu
from jax.experimental.pallas import tpu_sc as plsc

N = 10000
D = 128
G = 128
E = 320000

NP = 10240
E2 = 327680
SENT = 10000
NC = 2
NS = 16
CH = 128
CL = 64

f32 = jnp.float32
i32 = jnp.int32

def _sc_mesh():
    return plsc.VectorSubcoreMesh(
        core_axis_name="c", subcore_axis_name="s",
        num_cores=NC, num_subcores=NS)


def _k1_body(x_ref, w_ref, b_ref, bcol_ref, brow_ref, bprev_ref,
             h_ref, roots_ref, sizes_ref, isroot_ref):
    i = pl.program_id(0)
    h = jnp.dot(x_ref[...], w_ref[...], preferred_element_type=f32) + b_ref[...]
    h_ref[...] = jnp.where(h >= 0, h, 0.01 * h)

    bcol = jnp.broadcast_to(bcol_ref[...], (1000, G))
    lane = lax.broadcasted_iota(i32, (1000, G), 1)
    roots_p = jnp.sum((bcol < lane).astype(f32), axis=0, keepdims=True)
    sizes_p = jnp.sum((bcol == lane).astype(f32), axis=0, keepdims=True)

    @pl.when(i == 0)
    def _():
        roots_ref[...] = roots_p
        sizes_ref[...] = sizes_p

    @pl.when(i > 0)
    def _():
        roots_ref[...] += roots_p
        sizes_ref[...] += sizes_p

    isroot_ref[0] = (brow_ref[0] != bprev_ref[0]).astype(i32)


def _run_k1(x, adapter_w, adapter_b, batch):
    bcol = batch.reshape(N, 1)
    brow = batch.reshape(10, 1, 1000)
    bprev = jnp.concatenate([batch[:1] - 1, batch[:-1]]).reshape(10, 1, 1000)
    return pl.pallas_call(
        _k1_body,
        grid=(10,),
        in_specs=[
            pl.BlockSpec((1000, D), lambda i: (i, 0)),
            pl.BlockSpec((D, D), lambda i: (0, 0)),
            pl.BlockSpec((1, D), lambda i: (0, 0)),
            pl.BlockSpec((1000, 1), lambda i: (i, 0)),
            pl.BlockSpec((1, 1, 1000), lambda i: (i, 0, 0)),
            pl.BlockSpec((1, 1, 1000), lambda i: (i, 0, 0)),
        ],
        out_specs=[
            pl.BlockSpec((1000, D), lambda i: (i, 0)),
            pl.BlockSpec((1, G), lambda i: (0, 0)),
            pl.BlockSpec((1, G), lambda i: (0, 0)),
            pl.BlockSpec((1, 1, 1000), lambda i: (i, 0, 0)),
        ],
        out_shape=[
            jax.ShapeDtypeStruct((N, D), f32),
            jax.ShapeDtypeStruct((1, G), f32),
            jax.ShapeDtypeStruct((1, G), f32),
            jax.ShapeDtypeStruct((10, 1, 1000), i32),
        ],
    )(x, adapter_w, adapter_b.reshape(1, D), bcol, brow, bprev)


def _k2_body(src_hbm, dst_hbm, batch_hbm, isroot_hbm,
             pdegt_hbm, pdegb_hbm, pcnt_hbm,
             sidx, didx, batch_vm, isroot_vm, degt, degb, cnt):
    c = lax.axis_index("c")
    s = lax.axis_index("s")
    w = c * NS + s

    pltpu.sync_copy(src_hbm.at[w], sidx)
    pltpu.sync_copy(dst_hbm.at[w], didx)
    pltpu.sync_copy(batch_hbm, batch_vm)
    pltpu.sync_copy(isroot_hbm, isroot_vm)

    z16 = jnp.zeros((16,), f32)

    def zero_deg(j, _):
        degt[pl.ds(j * 16, 16)] = z16
        degb[pl.ds(j * 16, 16)] = z16
        return None
    lax.fori_loop(0, NP // 16, zero_deg, None)

    def zero_cnt(j, _):
        cnt[pl.ds(j * 16, 16)] = z16
        return None
    lax.fori_loop(0, 16, zero_cnt, None)

    ones = jnp.ones((16,), f32)
    one_i = jnp.ones((16,), i32)

    def chunk(t, _):
        for k in range(8):
            sv = sidx[t, pl.ds(k * 16, 16)]
            dv = didx[t, pl.ds(k * 16, 16)]
            bs = plsc.load_gather(batch_vm, [sv])
            bd = plsc.load_gather(batch_vm, [dv])
            rt = plsc.load_gather(isroot_vm, [sv])
            within = jnp.where((bs == bd) & (rt == one_i), 1.0, 0.0)
            plsc.addupdate_scatter(degt, [dv], ones)
            plsc.addupdate_scatter(degb, [sv], ones)
            plsc.addupdate_scatter(cnt, [bs], within)
        return None
    lax.fori_loop(0, 80, chunk, None)

    pltpu.sync_copy(degt, pdegt_hbm.at[w])
    pltpu.sync_copy(degb, pdegb_hbm.at[w])
    pltpu.sync_copy(cnt, pcnt_hbm.at[w])


def _run_k2(src_st, dst_st, batch_pad, isroot_pad):
    k = pl.kernel(
        _k2_body,
        out_type=[
            jax.ShapeDtypeStruct((32, NP), f32),
            jax.ShapeDtypeStruct((32, NP), f32),
            jax.ShapeDtypeStruct((32, 256), f32),
        ],
        mesh=_sc_mesh(),
        compiler_params=pltpu.CompilerParams(needs_layout_passes=False),
        scratch_types=[
            pltpu.VMEM((80, CH), i32),
            pltpu.VMEM((80, CH), i32),
            pltpu.VMEM((NP,), i32),
            pltpu.VMEM((NP,), i32),
            pltpu.VMEM((NP,), f32),
            pltpu.VMEM((NP,), f32),
            pltpu.VMEM((256,), f32),
        ],
    )
    return k(src_st, dst_st, batch_pad, isroot_pad)


def _ln_prompt(rf, w1, b1, g, be, w2, b2):
    t = jnp.dot(rf, w1, preferred_element_type=f32) + b1
    mu = jnp.mean(t, axis=-1, keepdims=True)
    var = jnp.mean((t - mu) ** 2, axis=-1, keepdims=True)
    t = (t - mu) * lax.rsqrt(var + 1e-5) * g + be
    return jnp.dot(jnp.tanh(t), w2, preferred_element_type=f32) + b2


def _k3a_body(pdegt_ref, pdegb_ref, pcnt_ref, sizes_ref, roots_ref,
              h_ref, brow_ref,
              p1w1, p1b1, p1g, p1be, p1w2, p1b2,
              p2w1, p2b1, p2g, p2be, p2w2, p2b2,
              dist_ref, disb_ref, pm_ref, pa_ref, alpha_ref, rf_acc):
    i = pl.program_id(0)
    degt = jnp.sum(pdegt_ref[...], axis=0, keepdims=True) + 1.0
    dist_ref[...] = lax.rsqrt(degt)
    degb = jnp.sum(pdegb_ref[...], axis=0, keepdims=True) + 1.0
    disb_ref[...] = lax.rsqrt(degb)

    gid = (lax.broadcasted_iota(i32, (G, 1000), 1) + i * 1000).astype(f32)
    r1h = jnp.broadcast_to(roots_ref[...], (G, 1000))
    rmat = (gid == r1h).astype(f32)
    rfp = jnp.dot(rmat, h_ref[...], preferred_element_type=f32)

    @pl.when(i == 0)
    def _():
        rf_acc[...] = rfp

    @pl.when(i > 0)
    def _():
        rf_acc[...] += rfp

    @pl.when(i == 9)
    def _():
        rf = rf_acc[...]
        pm_ref[...] = _ln_prompt(rf, p1w1[...], p1b1[...], p1g[...],
                                 p1be[...], p1w2[...], p1b2[...])
        pa_ref[...] = _ln_prompt(rf, p2w1[...], p2b1[...], p2g[...],
                                 p2be[...], p2w2[...], p2b2[...])
        cntv = jnp.sum(pcnt_ref[...], axis=0, keepdims=True)[:, :G]
        one_level = cntv / jnp.maximum(sizes_ref[...], 1.0)
        ag = jax.nn.sigmoid((one_level - 0.5) / 0.1)
        eye = (lax.broadcasted_iota(i32, (G, G), 0)
               == lax.broadcasted_iota(i32, (G, G), 1)).astype(f32)
        acol = lax.dot_general(eye, ag, (((0,), (1,)), ((), ())),
                               preferred_element_type=f32)
        alpha_ref[...] = jnp.broadcast_to(acol, (G, G))


def _run_k3a(pdegt, pdegb, pcnt, sizes, roots, h, batch, p):
    roots_col = roots.reshape(G, 1)
    brow = batch.reshape(10, 1, 1000)
    const = lambda i: (0, 0)
    pshape = [(D, 128), (1, 128), (1, 128), (1, 128), (128, D), (1, D)]
    pspecs = [pl.BlockSpec(s, const) for s in pshape] * 2
    pargs = [p['p1_W1'], p['p1_b1'].reshape(1, 128), p['p1_g'].reshape(1, 128),
             p['p1_be'].reshape(1, 128), p['p1_W2'], p['p1_b2'].reshape(1, D),
             p['p2_W1'], p['p2_b1'].reshape(1, 128), p['p2_g'].reshape(1, 128),
             p['p2_be'].reshape(1, 128), p['p2_W2'], p['p2_b2'].reshape(1, D)]
    return pl.pallas_call(
        _k3a_body,
        grid=(10,),
        in_specs=[
            pl.BlockSpec((32, 1024), lambda i: (0, i)),
            pl.BlockSpec((32, 1024), lambda i: (0, i)),
            pl.BlockSpec((32, 256), const),
            pl.BlockSpec((1, G), const),
            pl.BlockSpec((G, 1), const),
            pl.BlockSpec((1000, D), lambda i: (i, 0)),
            pl.BlockSpec((1, 1, 1000), lambda i: (i, 0, 0)),
        ] + pspecs,
        out_specs=[
            pl.BlockSpec((1, 1024), lambda i: (0, i)),
            pl.BlockSpec((1, 1024), lambda i: (0, i)),
            pl.BlockSpec((G, D), const),
            pl.BlockSpec((G, D), const),
            pl.BlockSpec((G, G), const),
        ],
        out_shape=[
            jax.ShapeDtypeStruct((1, NP), f32),
            jax.ShapeDtypeStruct((1, NP), f32),
            jax.ShapeDtypeStruct((G, D), f32),
            jax.ShapeDtypeStruct((G, D), f32),
            jax.ShapeDtypeStruct((G, G), f32),
        ],
        scratch_shapes=[pltpu.VMEM((G, D), f32)],
    )(pdegt, pdegb, pcnt, sizes, roots_col, h, brow, *pargs)


def _k3b_body(h_ref, brow_ref, dtc_ref, dbc_ref, pm_ref, pa_ref, alpha_ref,
              tdw_ref, buw_ref, mpt_ref, mpb_ref):
    bt = (lax.broadcasted_iota(i32, (G, 1000), 0)
          == jnp.broadcast_to(brow_ref[0], (G, 1000))).astype(f32)
    cdims = (((0,), (0,)), ((), ()))
    pm_n = lax.dot_general(bt, pm_ref[...], cdims, preferred_element_type=f32)
    pa_n = lax.dot_general(bt, pa_ref[...], cdims, preferred_element_type=f32)
    al_n = lax.dot_general(bt, alpha_ref[...], cdims, preferred_element_type=f32)
    h = h_ref[...]
    t1 = h * pm_n
    z = t1 + al_n * (h + pa_n - t1)
    mpt_ref[...] = jnp.dot(z, tdw_ref[...], preferred_element_type=f32) * dtc_ref[...]
    mpb_ref[...] = jnp.dot(z, buw_ref[...], preferred_element_type=f32) * dbc_ref[...]


def _run_k3b(h, batch, dist_col, disb_col, pm_g, pa_g, alpha_bcg, td_w1, bu_w1):
    const = lambda i: (0, 0)
    return pl.pallas_call(
        _k3b_body,
        grid=(10,),
        in_specs=[
            pl.BlockSpec((1000, D), lambda i: (i, 0)),
            pl.BlockSpec((1, 1, 1000), lambda i: (i, 0, 0)),
            pl.BlockSpec((1000, 1), lambda i: (i, 0)),
            pl.BlockSpec((1000, 1), lambda i: (i, 0)),
            pl.BlockSpec((G, D), const),
            pl.BlockSpec((G, D), const),
            pl.BlockSpec((G, G), const),
            pl.BlockSpec((D, D), const),
            pl.BlockSpec((D, D), const),
        ],
        out_specs=[
            pl.BlockSpec((1000, D), lambda i: (i, 0)),
            pl.BlockSpec((1000, D), lambda i: (i, 0)),
        ],
        out_shape=[
            jax.ShapeDtypeStruct((N, D), f32),
            jax.ShapeDtypeStruct((N, D), f32),
        ],
    )(h, batch.reshape(10, 1, 1000), dist_col, disb_col, pm_g, pa_g,
      alpha_bcg, td_w1, bu_w1)


def _layer_body(mp_hbm, gidx_hbm, didx_hbm, agg_hbm,
                sidxb, didxb, buf0, buf1, acc, is0, is1, gs0, gs1, ss0, ss1):
    c = lax.axis_index("c")
    s = lax.axis_index("s")

    z16 = jnp.zeros((16,), f32)

    def zero_buf(r, _):
        for k in range(8):
            buf0[r, pl.ds(k * 16, 16)] = z16
        return None
    lax.fori_loop(0, CH, zero_buf, None)

    def zero_acc(j, _):
        pltpu.sync_copy(buf0, acc.at[pl.ds(s * 640 + j * CH, CH)])
        return None
    lax.fori_loop(0, 5, zero_acc, None)
    plsc.subcore_barrier()

    bufs = (buf0, buf1)
    gsems = (gs0, gs1)
    ssems = (ss0, ss1)
    isems = (is0, is1)

    def idx_issue(blk, p):
        pltpu.async_copy(gidx_hbm.at[c, s, blk], sidxb.at[p], isems[p])
        pltpu.async_copy(didx_hbm.at[c, s, blk], didxb.at[p], isems[p])

    def idx_wait(blk, p):
        pltpu.make_async_copy(gidx_hbm.at[c, s, blk], sidxb.at[p], isems[p]).wait()
        pltpu.make_async_copy(didx_hbm.at[c, s, blk], didxb.at[p], isems[p]).wait()

    idx_issue(0, 0)
    idx_issue(1, 1)
    idx_wait(0, 0)
    pltpu.async_copy(mp_hbm.at[sidxb.at[0, 0]], buf0, gs0)
    pltpu.async_copy(mp_hbm.at[sidxb.at[0, 1]], buf1, gs1)

    def dblock(db, _):
        for p in range(2):
            b = 2 * db + p
            for j in range(16):
                q = j % 2
                if j == 2:
                    @pl.when((b > 0) & (b < 9))
                    def _():
                        idx_issue(b + 1, 1 - p)
                if j == 14:
                    @pl.when(b + 1 < 10)
                    def _():
                        idx_wait(b + 1, 1 - p)

                pltpu.make_async_copy(
                    mp_hbm.at[sidxb.at[p, j]], bufs[q], gsems[q]).wait()
                pltpu.async_copy(
                    bufs[q], acc.at[didxb.at[p, j]], ssems[q], add=True)
                pltpu.make_async_copy(
                    bufs[q], acc.at[didxb.at[0, 0]], ssems[q]).wait()
                if j < 14:
                    pltpu.async_copy(
                        mp_hbm.at[sidxb.at[p, j + 2]], bufs[q], gsems[q])
                else:
                    @pl.when(b < 9)
                    def _():
                        pltpu.async_copy(
                            mp_hbm.at[sidxb.at[1 - p, j - 14]], bufs[q],
                            gsems[q])
        return None
    lax.fori_loop(0, 5, dblock, None)

    plsc.subcore_barrier()

    def writeback(j, _):
        pltpu.sync_copy(acc.at[pl.ds(s * 640 + j * CH, CH)], buf0)
        pltpu.sync_copy(buf0, agg_hbm.at[c, pl.ds(s * 640 + j * CH, CH)])
        return None
    lax.fori_loop(0, 5, writeback, None)


def _run_layer(mp_all, gidx, didx):
    k = pl.kernel(
        _layer_body,
        out_type=jax.ShapeDtypeStruct((NC, NP, D), f32),
        mesh=_sc_mesh(),
        compiler_params=pltpu.CompilerParams(needs_layout_passes=False),
        scratch_types=[
            pltpu.VMEM((2, 16, CH), i32),
            pltpu.VMEM((2, 16, CH), i32),
            pltpu.VMEM((CH, D), f32),
            pltpu.VMEM((CH, D), f32),
            pltpu.VMEM_SHARED((NP, D), f32),
            pltpu.SemaphoreType.DMA,
            pltpu.SemaphoreType.DMA,
            pltpu.SemaphoreType.DMA,
            pltpu.SemaphoreType.DMA,
            pltpu.SemaphoreType.DMA,
            pltpu.SemaphoreType.DMA,
        ],
    )
    return k(mp_all, gidx, didx)


def _k5_body(aggt_ref, mpt_ref, aggb_ref, mpb_ref, dtc_ref, dbc_ref,
             tb1_ref, bb1_ref, tdw2_ref, buw2_ref, mpt2_ref, mpb2_ref):
    dt = dtc_ref[...]
    db = dbc_ref[...]
    h1t = dt * (aggt_ref[...] + mpt_ref[...]) + tb1_ref[...]
    h1t = jnp.maximum(h1t, 0.0)
    h1b = db * (aggb_ref[...] + mpb_ref[...]) + bb1_ref[...]
    h1b = jnp.maximum(h1b, 0.0)
    mpt2_ref[...] = jnp.dot(h1t, tdw2_ref[...], preferred_element_type=f32) * dt
    mpb2_ref[...] = jnp.dot(h1b, buw2_ref[...], preferred_element_type=f32) * db


def _run_k5(aggt, mpt, aggb, mpb, dist_col, disb_col, tb1, bb1, tdw2, buw2):
    const = lambda i: (0, 0)
    row = lambda i: (i, 0)
    return pl.pallas_call(
        _k5_body,
        grid=(10,),
        in_specs=[
            pl.BlockSpec((1000, D), row),
            pl.BlockSpec((1000, D), row),
            pl.BlockSpec((1000, D), row),
            pl.BlockSpec((1000, D), row),
            pl.BlockSpec((1000, 1), row),
            pl.BlockSpec((1000, 1), row),
            pl.BlockSpec((1, D), const),
            pl.BlockSpec((1, D), const),
            pl.BlockSpec((D, D), const),
            pl.BlockSpec((D, D), const),
        ],
        out_specs=[
            pl.BlockSpec((1000, D), row),
            pl.BlockSpec((1000, D), row),
        ],
        out_shape=[
            jax.ShapeDtypeStruct((N, D), f32),
            jax.ShapeDtypeStruct((N, D), f32),
        ],
    )(aggt, mpt, aggb, mpb, dist_col, disb_col,
      tb1.reshape(1, D), bb1.reshape(1, D), tdw2, buw2)


def _k7_body(aggt_ref, mpt_ref, aggb_ref, mpb_ref, dtc_ref, dbc_ref,
             brow_ref, tb2_ref, bb2_ref, pw1_ref, pb1_ref, pw2_ref, pb2_ref,
             out_ref, poolt, poolb):
    i = pl.program_id(0)
    q1 = dtc_ref[...] * (aggt_ref[...] + mpt_ref[...]) + tb2_ref[...]
    q2 = dbc_ref[...] * (aggb_ref[...] + mpb_ref[...]) + bb2_ref[...]
    bt = (lax.broadcasted_iota(i32, (G, 1000), 0)
          == jnp.broadcast_to(brow_ref[0], (G, 1000))).astype(f32)
    pt = jnp.dot(bt, q1, preferred_element_type=f32)
    pb = jnp.dot(bt, q2, preferred_element_type=f32)

    @pl.when(i == 0)
    def _():
        poolt[...] = pt
        poolb[...] = pb

    @pl.when(i > 0)
    def _():
        poolt[...] += pt
        poolb[...] += pb

    @pl.when(i == 9)
    def _():
        hc = jnp.concatenate([poolb[...], poolt[...]], axis=1)
        r = jnp.maximum(
            jnp.dot(hc, pw1_ref[...], preferred_element_type=f32) + pb1_ref[...],
            0.0)
        out_ref[...] = (jnp.dot(r, pw2_ref[...], preferred_element_type=f32)
                        + pb2_ref[...])


def _run_k7(aggt, mpt, aggb, mpb, dist_col, disb_col, batch, p):
    const = lambda i: (0, 0)
    row = lambda i: (i, 0)
    return pl.pallas_call(
        _k7_body,
        grid=(10,),
        in_specs=[
            pl.BlockSpec((1000, D), row),
            pl.BlockSpec((1000, D), row),
            pl.BlockSpec((1000, D), row),
            pl.BlockSpec((1000, D), row),
            pl.BlockSpec((1000, 1), row),
            pl.BlockSpec((1000, 1), row),
            pl.BlockSpec((1, 1, 1000), lambda i: (i, 0, 0)),
            pl.BlockSpec((1, D), const),
            pl.BlockSpec((1, D), const),
            pl.BlockSpec((2 * D, 256), const),
            pl.BlockSpec((1, 256), const),
            pl.BlockSpec((256, D), const),
            pl.BlockSpec((1, D), const),
        ],
        out_specs=pl.BlockSpec((G, D), const),
        out_shape=jax.ShapeDtypeStruct((G, D), f32),
        scratch_shapes=[pltpu.VMEM((G, D), f32), pltpu.VMEM((G, D), f32)],
    )(aggt, mpt, aggb, mpb, dist_col, disb_col, batch.reshape(10, 1, 1000),
      p['td_b2'].reshape(1, D), p['bu_b2'].reshape(1, D),
      p['proj_W1'], p['proj_b1'].reshape(1, 256),
      p['proj_W2'], p['proj_b2'].reshape(1, D))


def kernel(x, edge_index, batch, params):
    p = params
    src = edge_index[0]
    dst = edge_index[1]
    pad = jnp.full((E2 - E,), SENT, i32)
    srcp = jnp.concatenate([src, pad])
    dstp = jnp.concatenate([dst, pad])

    src_st = srcp.reshape(32, 80, CH)
    dst_st = dstp.reshape(32, 80, CH)
    gidx = jnp.stack([srcp, dstp + NP]).reshape(NC, NS, 10, 16, CH)
    didx = jnp.stack([dstp, srcp]).reshape(NC, NS, 10, 16, CH)

    batch_pad = jnp.concatenate([batch, jnp.full((NP - N,), G, i32)])

    h, roots, sizes, isroot = _run_k1(x, p['adapter_W'], p['adapter_b'], batch)
    isroot_pad = jnp.concatenate(
        [isroot.reshape(N), jnp.zeros((NP - N,), i32)])

    pdegt, pdegb, pcnt = _run_k2(src_st, dst_st, batch_pad, isroot_pad)

    dist_full, disb_full, pm_g, pa_g, alpha_bcg = _run_k3a(
        pdegt, pdegb, pcnt, sizes, roots, h, batch, p)
    dist_col = dist_full[0, :N].reshape(N, 1)
    disb_col = disb_full[0, :N].reshape(N, 1)

    mp1t, mp1b = _run_k3b(h, batch, dist_col, disb_col, pm_g, pa_g,
                          alpha_bcg, p['td_W1'], p['bu_W1'])

    zpad = jnp.zeros((NP - N, D), f32)
    mp_all1 = jnp.concatenate([mp1t, zpad, mp1b, zpad]).reshape(NC * NP, D)
    agg1 = _run_layer(mp_all1, gidx, didx)

    mp2t, mp2b = _run_k5(agg1[0, :N], mp1t, agg1[1, :N], mp1b,
                         dist_col, disb_col, p['td_b1'], p['bu_b1'],
                         p['td_W2'], p['bu_W2'])

    mp_all2 = jnp.concatenate([mp2t, zpad, mp2b, zpad]).reshape(NC * NP, D)
    agg2 = _run_layer(mp_all2, gidx, didx)

    return _run_k7(agg2[0, :N], mp2t, agg2[1, :N], mp2b,
                   dist_col, disb_col, batch, p)
